# baseline (speedup 1.0000x reference)
import jax
import jax.numpy as jnp
from jax import lax
from jax.experimental import pallas as pl
from jax.experimental.pallas import tpu as pltpu
from jax.experimental.pallas import tpu_sc as plsc

N_CLASS = 100000
FEA_DIM = 128
N_SAMPLES = 16384
L = 16
NF = FEA_DIM // L

NC = 2
NS = 16
NW = NC * NS
ROWS_PER_W = N_CLASS // NW
CB = 384
NCB = -(-ROWS_PER_W // CB)
NBUF = 2
EB = 96
LOG2_N = 14


def _sread(ref, i):
    return ref[pl.ds(i, L)][0]


def _lower_bound(ids_ref, limit):
    def body(_, c):
        lo, hi = c
        mid = (lo + hi) // 2
        pred = _sread(ids_ref, mid) < limit
        return (jnp.where(pred, mid + 1, lo), jnp.where(pred, hi, mid))
    lo, _ = lax.fori_loop(0, LOG2_N, body,
                          (jnp.int32(0), jnp.int32(N_SAMPLES)))
    return lo


def _base(r0, b):
    return r0 + jnp.minimum(b * CB, ROWS_PER_W - CB)


def _sc_body(pp_hbm, emb_hbm, ids_hbm, out_hbm,
             ids_v, bufs, emb_buf, ld_sem, st_sem):
    wid = lax.axis_index("s") * NC + lax.axis_index("c")
    r0 = wid * ROWS_PER_W
    lanes = lax.iota(jnp.int32, L)

    def splat(x):
        return jnp.full((L,), x, jnp.int32)

    pltpu.sync_copy(ids_hbm, ids_v.at[pl.ds(0, N_SAMPLES)])

    lo0 = _lower_bound(ids_v, r0)
    pltpu.async_copy(pp_hbm.at[pl.ds(r0, CB)], bufs.at[0], ld_sem)

    zrow = tuple(jnp.zeros((L,), jnp.float32) for _ in range(NF))

    def block(b, lo_b):
        cur = lax.rem(b, NBUF)
        nxt = lax.rem(b + 1, NBUF)
        base = _base(r0, b)
        hi_b = _lower_bound(ids_v, base + CB)

        pltpu.make_async_copy(pp_hbm.at[pl.ds(base, CB)], bufs.at[cur],
                              ld_sem).wait()

        @pl.when(b + 1 < NCB)
        def _():
            @pl.when(b >= NBUF - 1)
            def _():
                pltpu.make_async_copy(bufs.at[nxt],
                                      out_hbm.at[pl.ds(base, CB)],
                                      st_sem).wait()
            pltpu.async_copy(pp_hbm.at[pl.ds(_base(r0, b + 1), CB)],
                             bufs.at[nxt], ld_sem)

        def apply_mean(seg_id, acc, cnt):
            rcv = jnp.full((L,), 1.0, jnp.float32) / jnp.full((L,), cnt,
                                                              jnp.float32)
            row = splat(seg_id - base)
            for k in range(NF):
                plsc.store_scatter(bufs, [splat(cur), row, k * L + lanes],
                                   acc[k] * rcv)

        n_b = hi_b - lo_b
        nscan = (n_b + EB - 1) // EB

        def scan_outer(e, carry):
            start = lo_b + e * EB
            start_c = jnp.minimum(start, N_SAMPLES - EB)
            blk_end = jnp.minimum(start + EB, hi_b)
            pltpu.sync_copy(
                emb_hbm.at[pl.ds(start_c * FEA_DIM, EB * FEA_DIM)], emb_buf)

            def inner(j, c):
                acc, cnt, prev = c
                idj = _sread(ids_v, j)
                loc = j - start_c
                row = tuple(emb_buf[pl.ds(loc * FEA_DIM + k * L, L)]
                            for k in range(NF))
                is_new = idj != prev

                @pl.when(jnp.logical_and(is_new, cnt > 0.0))
                def _():
                    apply_mean(prev, acc, cnt)

                acc = tuple(jnp.where(is_new, row[k], acc[k] + row[k])
                            for k in range(NF))
                cnt = jnp.where(is_new, jnp.float32(1.0), cnt + 1.0)
                return (acc, cnt, idj)

            return lax.fori_loop(start, blk_end, inner, carry)

        init = (zrow, jnp.float32(0.0), jnp.int32(-1))
        acc, cnt, prev = lax.fori_loop(0, nscan, scan_outer, init)

        @pl.when(jnp.logical_and(n_b > 0, cnt > 0.0))
        def _():
            apply_mean(prev, acc, cnt)

        lo_next = _lower_bound(ids_v, _base(r0, b + 1))

        pltpu.async_copy(bufs.at[cur], out_hbm.at[pl.ds(base, CB)], st_sem)
        return lo_next

    lax.fori_loop(0, NCB, block, lo0)

    for bb in range(NCB - NBUF, NCB):
        pltpu.make_async_copy(bufs.at[lax.rem(jnp.int32(bb), NBUF)],
                              out_hbm.at[pl.ds(_base(r0, bb), CB)],
                              st_sem).wait()


def kernel(pp_running, embeddings, class_ids):
    ids = class_ids.astype(jnp.int32)
    emb_flat = embeddings.reshape(N_SAMPLES * FEA_DIM)
    mesh = plsc.VectorSubcoreMesh(core_axis_name="c", subcore_axis_name="s")
    f = pl.kernel(
        _sc_body,
        out_type=jax.ShapeDtypeStruct((N_CLASS, FEA_DIM), jnp.float32),
        mesh=mesh,
        compiler_params=pltpu.CompilerParams(use_tc_tiling_on_sc=False,
                                             needs_layout_passes=False),
        scratch_types=[
            pltpu.VMEM((N_SAMPLES + L,), jnp.int32),
            pltpu.VMEM((NBUF, CB, FEA_DIM), jnp.float32),
            pltpu.VMEM((EB * FEA_DIM,), jnp.float32),
            pltpu.SemaphoreType.DMA,
            pltpu.SemaphoreType.DMA,
        ],
    )
    return f(pp_running, emb_flat, ids)


# speedup vs baseline: 1.1744x; 1.1744x over previous
import jax
import jax.numpy as jnp
from jax import lax
from jax.experimental import pallas as pl
from jax.experimental.pallas import tpu as pltpu
from jax.experimental.pallas import tpu_sc as plsc

N_CLASS = 100000
FEA_DIM = 128
N_SAMPLES = 16384
L = 16
NF = FEA_DIM // L

NC = 2
NS = 16
NW = NC * NS
ROWS_PER_W = N_CLASS // NW
CB = 250
NCB = -(-ROWS_PER_W // CB)
NBUF = 3
EB = 64
LOG2_N = 14


def _sread(ref, i):
    return ref[pl.ds(i, L)][0]


def _lower_bound(ids_ref, limit):
    def body(_, c):
        lo, hi = c
        mid = (lo + hi) // 2
        pred = _sread(ids_ref, mid) < limit
        return (jnp.where(pred, mid + 1, lo), jnp.where(pred, hi, mid))
    lo, _ = lax.fori_loop(0, LOG2_N, body,
                          (jnp.int32(0), jnp.int32(N_SAMPLES)))
    return lo


def _base(r0, b):
    return r0 + jnp.minimum(b * CB, ROWS_PER_W - CB)


def _sc_body(pp_hbm, emb_hbm, ids_hbm, out_hbm,
             ids_v, bufs, emb_buf, ld_sem, st_sem, eb_sem):
    wid = lax.axis_index("s") * NC + lax.axis_index("c")
    r0 = wid * ROWS_PER_W
    lanes = lax.iota(jnp.int32, L)
    EBW = EB * FEA_DIM

    def splat(x):
        return jnp.full((L,), x, jnp.int32)

    def emb_start(lo, which, sem):
        start_c = jnp.minimum(lo, N_SAMPLES - EB)
        return pltpu.async_copy(
            emb_hbm.at[pl.ds(start_c * FEA_DIM, EBW)],
            emb_buf.at[pl.ds(which * EBW, EBW)], sem)

    pltpu.sync_copy(ids_hbm, ids_v.at[pl.ds(0, N_SAMPLES)])

    lo0 = _lower_bound(ids_v, r0)
    pltpu.async_copy(pp_hbm.at[pl.ds(r0, CB)], bufs.at[0], ld_sem)
    emb_start(lo0, jnp.int32(0), eb_sem)

    zrow = tuple(jnp.zeros((L,), jnp.float32) for _ in range(NF))

    def block(b, lo_b):
        cur = lax.rem(b, NBUF)
        nxt = lax.rem(b + 1, NBUF)
        base = _base(r0, b)
        hi_b = _lower_bound(ids_v, base + CB)

        pltpu.make_async_copy(pp_hbm.at[pl.ds(base, CB)], bufs.at[cur],
                              ld_sem).wait()

        @pl.when(b + 1 < NCB)
        def _():
            @pl.when(b >= NBUF - 1)
            def _():
                pltpu.make_async_copy(bufs.at[nxt],
                                      out_hbm.at[pl.ds(base, CB)],
                                      st_sem).wait()
            pltpu.async_copy(pp_hbm.at[pl.ds(_base(r0, b + 1), CB)],
                             bufs.at[nxt], ld_sem)

        def apply_mean(seg_id, acc, cnt):
            rcv = jnp.full((L,), 1.0, jnp.float32) / jnp.full((L,), cnt,
                                                              jnp.float32)
            row = splat(seg_id - base)
            for k in range(NF):
                plsc.store_scatter(bufs, [splat(cur), row, k * L + lanes],
                                   acc[k] * rcv)

        n_b = hi_b - lo_b
        nscan = (n_b + EB - 1) // EB

        pltpu.make_async_copy(emb_hbm.at[pl.ds(0, EB * FEA_DIM)],
                              emb_buf.at[pl.ds(0, EB * FEA_DIM)],
                              eb_sem).wait()

        def scan_outer(e, carry):
            start = lo_b + e * EB
            start_c = jnp.minimum(start, N_SAMPLES - EB)
            blk_end = jnp.minimum(start + EB, hi_b)
            which = lax.rem(b + e, 2)

            @pl.when(e >= 1)
            def _():
                pltpu.sync_copy(
                    emb_hbm.at[pl.ds(start_c * FEA_DIM, EB * FEA_DIM)],
                    emb_buf.at[pl.ds(which * EB * FEA_DIM,
                                     EB * FEA_DIM)])

            ebase = which * EB * FEA_DIM

            def inner(j, c):
                acc, cnt, prev = c
                idj = _sread(ids_v, j)
                loc = j - start_c
                row = tuple(emb_buf[pl.ds(ebase + loc * FEA_DIM + k * L,
                                          L)]
                            for k in range(NF))
                is_new = idj != prev

                @pl.when(jnp.logical_and(is_new, cnt > 0.0))
                def _():
                    apply_mean(prev, acc, cnt)

                acc = tuple(jnp.where(is_new, row[k], acc[k] + row[k])
                            for k in range(NF))
                cnt = jnp.where(is_new, jnp.float32(1.0), cnt + 1.0)
                return (acc, cnt, idj)

            return lax.fori_loop(start, blk_end, inner, carry)

        init = (zrow, jnp.float32(0.0), jnp.int32(-1))
        acc, cnt, prev = lax.fori_loop(0, nscan, scan_outer, init)

        @pl.when(jnp.logical_and(n_b > 0, cnt > 0.0))
        def _():
            apply_mean(prev, acc, cnt)

        lo_next = _lower_bound(ids_v, _base(r0, b + 1))

        pltpu.async_copy(bufs.at[cur], out_hbm.at[pl.ds(base, CB)], st_sem)

        @pl.when(b + 1 < NCB)
        def _():
            emb_start(lo_next, lax.rem(b + 1, 2), eb_sem)
        return lo_next

    lax.fori_loop(0, NCB, block, lo0)

    for bb in range(NCB - NBUF, NCB):
        pltpu.make_async_copy(bufs.at[lax.rem(jnp.int32(bb), NBUF)],
                              out_hbm.at[pl.ds(_base(r0, bb), CB)],
                              st_sem).wait()


def kernel(pp_running, embeddings, class_ids):
    ids = class_ids.astype(jnp.int32)
    emb_flat = embeddings.reshape(N_SAMPLES * FEA_DIM)
    mesh = plsc.VectorSubcoreMesh(core_axis_name="c", subcore_axis_name="s")
    f = pl.kernel(
        _sc_body,
        out_type=jax.ShapeDtypeStruct((N_CLASS, FEA_DIM), jnp.float32),
        mesh=mesh,
        compiler_params=pltpu.CompilerParams(use_tc_tiling_on_sc=False,
                                             needs_layout_passes=False),
        scratch_types=[
            pltpu.VMEM((N_SAMPLES + L,), jnp.int32),
            pltpu.VMEM((NBUF, CB, FEA_DIM), jnp.float32),
            pltpu.VMEM((2 * EB * FEA_DIM,), jnp.float32),
            pltpu.SemaphoreType.DMA,
            pltpu.SemaphoreType.DMA,
            pltpu.SemaphoreType.DMA,
        ],
    )
    return f(pp_running, emb_flat, ids)
